# baseline (speedup 1.0000x reference)
import functools

import jax
import jax.numpy as jnp
from jax import lax
from jax.experimental import pallas as pl
from jax.experimental.pallas import tpu as pltpu
from jax.experimental.pallas import tpu_sc as plsc

N = 10000
E = 320000
F = 128
NSUB = 16
B = 32
OCT = 8
RPT = (N // NSUB) // 8 * 8
TAIL = N - RPT * NSUB
NREG = F // 16


def _mlp_body(q_ref, w1_ref, b1_ref, w2_ref, b2_ref,
              mu0_ref, mu1_ref, mu2_ref,
              x0_ref, x1_ref, p0_ref, p1_ref, p2_ref):
    h = jnp.dot(q_ref[...], w1_ref[...], preferred_element_type=jnp.float32)
    h = h + b1_ref[...]
    h = h * jax.nn.sigmoid(h)
    x = jnp.dot(h, w2_ref[...], preferred_element_type=jnp.float32) + b2_ref[...]
    x0_ref[...] = x[:, :F]
    x1_ref[...] = x[:, F:2 * F]
    x2 = x[:, 2 * F:]
    p0_ref[...] = x2 * mu0_ref[...]
    p1_ref[...] = x2 * mu1_ref[...]
    p2_ref[...] = x2 * mu2_ref[...]


def _mlp(q2, W1, b1, W2, b2, mu0, mu1, mu2):
    BN = 1000
    grid = (N // BN,)
    blk = pl.BlockSpec((BN, F), lambda i: (i, 0))
    out = pl.pallas_call(
        _mlp_body,
        grid=grid,
        in_specs=[
            blk,
            pl.BlockSpec((F, F), lambda i: (0, 0)),
            pl.BlockSpec((1, F), lambda i: (0, 0)),
            pl.BlockSpec((F, 3 * F), lambda i: (0, 0)),
            pl.BlockSpec((1, 3 * F), lambda i: (0, 0)),
            blk, blk, blk,
        ],
        out_specs=[blk] * 5,
        out_shape=[jax.ShapeDtypeStruct((N, F), jnp.float32)] * 5,
    )(q2, W1, b1.reshape(1, F), W2, b2.reshape(1, 3 * F), mu0, mu1, mu2)
    return out


def _sc_edge_kernel(w_hbm, x0_hbm, x1_hbm, pcat_hbm, dcat_hbm,
                    q2_hbm, mucat_hbm, idxi_hbm, idxj_hbm,
                    outq_hbm, outmu_hbm,
                    acc, *scr):
    cid = lax.axis_index("c")
    sid = lax.axis_index("s")
    row0 = sid * RPT
    ebase = sid * 624 * B
    nchunk = 624 + jnp.where(sid == 15, 16, 0)

    idxj_o = scr[0:2]
    idxi_o = scr[2:4]
    idxp_o = scr[4:6]
    dir_o = scr[6:8]
    idxs_v = scr[8:10]
    data = (scr[10:14], scr[14:18])
    sem_oct = scr[18:20]
    sem_load = scr[20:22]
    sem_scat = scr[22:24]

    def init_writeout(hbm, rowoff, to_acc):
        if not to_acc:
            plsc.subcore_barrier()

        def cp(src, dst):
            pltpu.sync_copy(src, dst)

        a = acc.at[pl.ds(row0, RPT), :]
        h = hbm.at[pl.ds(rowoff + row0, RPT), :]
        cp(h, a) if to_acc else cp(a, h)

        @pl.when(sid == NSUB - 1)
        def _():
            a2 = acc.at[pl.ds(RPT * NSUB, TAIL), :]
            h2 = hbm.at[pl.ds(rowoff + RPT * NSUB, TAIL), :]
            cp(h2, a2) if to_acc else cp(a2, h2)

        plsc.subcore_barrier()

    def run_pass(is_dq, xa_hbm, xb_hbm, base_hbm, out_hbm, d):
        rowoff = d * N
        init_writeout(base_hbm, rowoff, True)

        def issue_oct(par, koct0):
            e0 = ebase + koct0 * B
            pltpu.async_copy(idxj_hbm.at[pl.ds(e0, OCT * B)], idxj_o[par],
                             sem_oct[par])
            pltpu.async_copy(idxi_hbm.at[pl.ds(e0, OCT * B)], idxi_o[par],
                             sem_oct[par])
            if not is_dq:
                pltpu.async_copy(dcat_hbm.at[pl.ds(d * E + e0, OCT * B)],
                                 dir_o[par], sem_oct[par])

        def wait_oct(par):
            pltpu.make_async_copy(idxj_hbm.at[pl.ds(0, OCT * B)],
                                  idxj_o[par], sem_oct[par]).wait()
            pltpu.make_async_copy(idxi_hbm.at[pl.ds(0, OCT * B)],
                                  idxi_o[par], sem_oct[par]).wait()
            if not is_dq:
                pltpu.make_async_copy(dcat_hbm.at[pl.ds(0, OCT * B)],
                                      dir_o[par], sem_oct[par]).wait()

        def issue_gathers(s, par, p, k):
            w12_v, ga_v, gb_v, _ = data[s]
            e0 = ebase + k * B
            ij = idxj_o[par].at[pl.ds(p * B, B)]
            if is_dq:
                pltpu.async_copy(w_hbm.at[pl.ds(e0, B), pl.ds(0, F)],
                                 ga_v, sem_load[s])
                pltpu.async_copy(xb_hbm.at[ij], gb_v, sem_load[s])
            else:
                for t in range(B // 16):
                    sl = pl.ds(p * B + t * 16, 16)
                    idxp_o[par][sl] = idxj_o[par][sl] + rowoff
                ip = idxp_o[par].at[pl.ds(p * B, B)]
                pltpu.async_copy(w_hbm.at[pl.ds(e0, B), pl.ds(F, 2 * F)],
                                 w12_v, sem_load[s])
                pltpu.async_copy(xa_hbm.at[ij], ga_v, sem_load[s])
                pltpu.async_copy(xb_hbm.at[ip], gb_v, sem_load[s])

        def wait_loads(s, par, p):
            w12_v, ga_v, gb_v, _ = data[s]
            ij = idxj_o[par].at[pl.ds(p * B, B)]
            if is_dq:
                pltpu.make_async_copy(w_hbm.at[pl.ds(0, B), pl.ds(0, F)],
                                      ga_v, sem_load[s]).wait()
                pltpu.make_async_copy(xb_hbm.at[ij], gb_v,
                                      sem_load[s]).wait()
            else:
                ip = idxp_o[par].at[pl.ds(p * B, B)]
                pltpu.make_async_copy(w_hbm.at[pl.ds(0, B), pl.ds(F, 2 * F)],
                                      w12_v, sem_load[s]).wait()
                pltpu.make_async_copy(xa_hbm.at[ij], ga_v,
                                      sem_load[s]).wait()
                pltpu.make_async_copy(xb_hbm.at[ip], gb_v,
                                      sem_load[s]).wait()

        def compute(s, par, p):
            w12_v, ga_v, gb_v, out_v = data[s]
            if is_dq:
                @pl.loop(0, B, unroll=2)
                def _edge(e):
                    for r in range(NREG):
                        sl = pl.ds(r * 16, 16)
                        out_v[e, sl] = ga_v[e, sl] * gb_v[e, sl]
            else:
                dv = dir_o[par]

                @pl.loop(0, B, unroll=2)
                def _edge(e):
                    esplat = jnp.broadcast_to(e + p * B, (16,)).astype(
                        jnp.int32)
                    dval = plsc.load_gather(dv, [esplat])
                    for r in range(NREG):
                        sl = pl.ds(r * 16, 16)
                        sl2 = pl.ds(F + r * 16, 16)
                        out_v[e, sl] = (w12_v[e, sl] * ga_v[e, sl] * dval
                                        + w12_v[e, sl2] * gb_v[e, sl])

        def issue_scatter(s, par, p):
            out_v = data[s][3]
            for t in range(B // 16):
                idxs_v[s][pl.ds(t * 16, 16)] = \
                    idxi_o[par][pl.ds(p * B + t * 16, 16)]
            pltpu.async_copy(out_v, acc.at[idxs_v[s]], sem_scat[s], add=True)

        def wait_scatter(s):
            out_v = data[s][3]
            pltpu.make_async_copy(out_v, acc.at[idxs_v[s]],
                                  sem_scat[s]).wait()

        issue_oct(0, 0)
        wait_oct(0)
        issue_gathers(0, 0, 0, 0)
        issue_gathers(1, 0, 1, 1)

        @pl.loop(0, nchunk // (2 * OCT))
        def _pair(mm):
            for oo in (0, 1):
                par = oo
                k0 = (2 * mm + oo) * OCT
                for p in range(OCT):
                    k = k0 + p
                    s = p % 2

                    wait_loads(s, par, p)

                    @pl.when(k >= 2)
                    def _():
                        wait_scatter(s)

                    compute(s, par, p)
                    issue_scatter(s, par, p)

                    if p == 2:
                        @pl.when(k0 + OCT < nchunk)
                        def _():
                            issue_oct(1 - par, k0 + OCT)

                    if p == 6:
                        @pl.when(k0 + OCT < nchunk)
                        def _():
                            wait_oct(1 - par)

                    @pl.when(k + 2 < nchunk)
                    def _():
                        issue_gathers(s, par if p < 6 else 1 - par,
                                      (p + 2) % OCT, k + 2)

        wait_scatter(0)
        wait_scatter(1)
        init_writeout(out_hbm, rowoff, False)

    @pl.when(cid == 0)
    def _():
        run_pass(True, x0_hbm, x0_hbm, q2_hbm, outq_hbm,
                 jnp.zeros((), jnp.int32))

    dlo = jnp.where(cid == 0, 0, 1)
    dhi = jnp.where(cid == 0, 1, 3)

    @pl.loop(dlo, dhi)
    def _dmu(d):
        run_pass(False, x1_hbm, pcat_hbm, mucat_hbm, outmu_hbm, d)


_sc_edges = functools.partial(
    pl.kernel,
    out_type=[jax.ShapeDtypeStruct((N, F), jnp.float32),
              jax.ShapeDtypeStruct((3 * N, F), jnp.float32)],
    mesh=plsc.VectorSubcoreMesh(core_axis_name="c", subcore_axis_name="s"),
    compiler_params=pltpu.CompilerParams(needs_layout_passes=False),
    scratch_types=[pltpu.VMEM_SHARED((N, F), jnp.float32)]
    + [pltpu.VMEM((OCT * B,), jnp.int32)] * 2
    + [pltpu.VMEM((OCT * B,), jnp.int32)] * 2
    + [pltpu.VMEM((OCT * B,), jnp.int32)] * 2
    + [pltpu.VMEM((OCT * B,), jnp.float32)] * 2
    + [pltpu.VMEM((B,), jnp.int32)] * 2
    + [pltpu.VMEM((B, 2 * F), jnp.float32),
       pltpu.VMEM((B, F), jnp.float32),
       pltpu.VMEM((B, F), jnp.float32),
       pltpu.VMEM((B, F), jnp.float32)]
    + [pltpu.VMEM((B, 2 * F), jnp.float32),
       pltpu.VMEM((B, F), jnp.float32),
       pltpu.VMEM((B, F), jnp.float32),
       pltpu.VMEM((B, F), jnp.float32)]
    + [pltpu.SemaphoreType.DMA] * 2
    + [pltpu.SemaphoreType.DMA] * 4,
)(_sc_edge_kernel)


def kernel(q, mu, W_ij, dir_ij, pairlist, W1, b1, W2, b2):
    q2 = q[:, 0, :]
    mu0, mu1, mu2 = mu[:, 0, :], mu[:, 1, :], mu[:, 2, :]
    x0, x1, p0, p1, p2 = _mlp(q2, W1, b1, W2, b2, mu0, mu1, mu2)
    pcat = jnp.concatenate([p0, p1, p2], axis=0)
    mucat = jnp.concatenate([mu0, mu1, mu2], axis=0)
    dcat = jnp.concatenate([dir_ij[:, 0], dir_ij[:, 1], dir_ij[:, 2]], axis=0)
    idx_i, idx_j = pairlist[0], pairlist[1]
    outq, outmu = _sc_edges(W_ij, x0, x1, pcat, dcat,
                            q2, mucat, idx_i, idx_j)
    return outq[:, None, :], outmu.reshape(3, N, F).transpose(1, 0, 2)


# speedup vs baseline: 1.0631x; 1.0631x over previous
import functools

import jax
import jax.numpy as jnp
from jax import lax
from jax.experimental import pallas as pl
from jax.experimental.pallas import tpu as pltpu
from jax.experimental.pallas import tpu_sc as plsc

N = 10000
E = 320000
F = 128
NSUB = 16
B = 32
OCT = 8
RPT = (N // NSUB) // 8 * 8
TAIL = N - RPT * NSUB
NREG = F // 16


def _mlp_body(q_ref, w1_ref, b1_ref, w2_ref, b2_ref,
              mu0_ref, mu1_ref, mu2_ref,
              x0_ref, x1_ref, p0_ref, p1_ref, p2_ref):
    h = jnp.dot(q_ref[...], w1_ref[...], preferred_element_type=jnp.float32)
    h = h + b1_ref[...]
    h = h * jax.nn.sigmoid(h)
    x = jnp.dot(h, w2_ref[...], preferred_element_type=jnp.float32) + b2_ref[...]
    x0_ref[...] = x[:, :F]
    x1_ref[...] = x[:, F:2 * F]
    x2 = x[:, 2 * F:]
    p0_ref[...] = x2 * mu0_ref[...]
    p1_ref[...] = x2 * mu1_ref[...]
    p2_ref[...] = x2 * mu2_ref[...]


def _mlp(q2, W1, b1, W2, b2, mu0, mu1, mu2):
    BN = 1000
    grid = (N // BN,)
    blk = pl.BlockSpec((BN, F), lambda i: (i, 0))
    out = pl.pallas_call(
        _mlp_body,
        grid=grid,
        in_specs=[
            blk,
            pl.BlockSpec((F, F), lambda i: (0, 0)),
            pl.BlockSpec((1, F), lambda i: (0, 0)),
            pl.BlockSpec((F, 3 * F), lambda i: (0, 0)),
            pl.BlockSpec((1, 3 * F), lambda i: (0, 0)),
            blk, blk, blk,
        ],
        out_specs=[blk] * 5,
        out_shape=[jax.ShapeDtypeStruct((N, F), jnp.float32)] * 5,
    )(q2, W1, b1.reshape(1, F), W2, b2.reshape(1, 3 * F), mu0, mu1, mu2)
    return out


CSPLIT = 96


def _sc_edge_kernel(w_hbm, x0_hbm, x1_hbm, pcat_hbm, dcat_hbm,
                    q2_hbm, mucat_hbm, zq_hbm, idxi_hbm, idxj_hbm,
                    outq_hbm, outmu_hbm, outp_hbm,
                    acc, *scr):
    cid = lax.axis_index("c")
    sid = lax.axis_index("s")
    row0 = sid * RPT
    ebase = sid * 624 * B
    nchunk = 624 + jnp.where(sid == 15, 16, 0)

    idxj_o = scr[0:2]
    idxi_o = scr[2:4]
    idxp_o = scr[4:6]
    dir_o = scr[6:8]
    idxs_v = scr[8:10]
    data = (scr[10:14], scr[14:18])
    sem_oct = scr[18:20]
    sem_load = scr[20:22]
    sem_scat = scr[22:24]

    def init_writeout(hbm, alt_hbm, use_alt, rowoff, to_acc):
        if not to_acc:
            plsc.subcore_barrier()

        def cp(src, dst):
            pltpu.sync_copy(src, dst)

        def body(h_hbm, roff):
            a = acc.at[pl.ds(row0, RPT), :]
            h = h_hbm.at[pl.ds(roff + row0, RPT), :]
            cp(h, a) if to_acc else cp(a, h)

            @pl.when(sid == NSUB - 1)
            def _():
                a2 = acc.at[pl.ds(RPT * NSUB, TAIL), :]
                h2 = h_hbm.at[pl.ds(roff + RPT * NSUB, TAIL), :]
                cp(h2, a2) if to_acc else cp(a2, h2)

        @pl.when(use_alt)
        def _():
            body(alt_hbm, 0)

        @pl.when(jnp.logical_not(use_alt))
        def _():
            body(hbm, rowoff)

        plsc.subcore_barrier()

    def run_pass(is_dq, xa_hbm, xb_hbm, base_hbm, out_hbm, d,
                 clo_p, chi_p, use_alt):
        rowoff = d * N
        kstart = clo_p * 2 * OCT
        init_writeout(base_hbm, zq_hbm, use_alt, rowoff, True)

        def issue_oct(par, koct0):
            e0 = ebase + koct0 * B
            pltpu.async_copy(idxj_hbm.at[pl.ds(e0, OCT * B)], idxj_o[par],
                             sem_oct[par])
            pltpu.async_copy(idxi_hbm.at[pl.ds(e0, OCT * B)], idxi_o[par],
                             sem_oct[par])
            if not is_dq:
                pltpu.async_copy(dcat_hbm.at[pl.ds(d * E + e0, OCT * B)],
                                 dir_o[par], sem_oct[par])

        def wait_oct(par):
            pltpu.make_async_copy(idxj_hbm.at[pl.ds(0, OCT * B)],
                                  idxj_o[par], sem_oct[par]).wait()
            pltpu.make_async_copy(idxi_hbm.at[pl.ds(0, OCT * B)],
                                  idxi_o[par], sem_oct[par]).wait()
            if not is_dq:
                pltpu.make_async_copy(dcat_hbm.at[pl.ds(0, OCT * B)],
                                      dir_o[par], sem_oct[par]).wait()

        def issue_gathers(s, par, p, k):
            w12_v, ga_v, gb_v, _ = data[s]
            e0 = ebase + k * B
            ij = idxj_o[par].at[pl.ds(p * B, B)]
            if is_dq:
                pltpu.async_copy(w_hbm.at[pl.ds(e0, B), pl.ds(0, F)],
                                 ga_v, sem_load[s])
                pltpu.async_copy(xb_hbm.at[ij], gb_v, sem_load[s])
            else:
                for t in range(B // 16):
                    sl = pl.ds(p * B + t * 16, 16)
                    idxp_o[par][sl] = idxj_o[par][sl] + rowoff
                ip = idxp_o[par].at[pl.ds(p * B, B)]
                pltpu.async_copy(w_hbm.at[pl.ds(e0, B), pl.ds(F, 2 * F)],
                                 w12_v, sem_load[s])
                pltpu.async_copy(xa_hbm.at[ij], ga_v, sem_load[s])
                pltpu.async_copy(xb_hbm.at[ip], gb_v, sem_load[s])

        def wait_loads(s, par, p):
            w12_v, ga_v, gb_v, _ = data[s]
            ij = idxj_o[par].at[pl.ds(p * B, B)]
            if is_dq:
                pltpu.make_async_copy(w_hbm.at[pl.ds(0, B), pl.ds(0, F)],
                                      ga_v, sem_load[s]).wait()
                pltpu.make_async_copy(xb_hbm.at[ij], gb_v,
                                      sem_load[s]).wait()
            else:
                ip = idxp_o[par].at[pl.ds(p * B, B)]
                pltpu.make_async_copy(w_hbm.at[pl.ds(0, B), pl.ds(F, 2 * F)],
                                      w12_v, sem_load[s]).wait()
                pltpu.make_async_copy(xa_hbm.at[ij], ga_v,
                                      sem_load[s]).wait()
                pltpu.make_async_copy(xb_hbm.at[ip], gb_v,
                                      sem_load[s]).wait()

        def compute(s, par, p):
            w12_v, ga_v, gb_v, out_v = data[s]
            if is_dq:
                @pl.loop(0, B, unroll=2)
                def _edge(e):
                    for r in range(NREG):
                        sl = pl.ds(r * 16, 16)
                        out_v[e, sl] = ga_v[e, sl] * gb_v[e, sl]
            else:
                dv = dir_o[par]

                @pl.loop(0, B, unroll=2)
                def _edge(e):
                    esplat = jnp.broadcast_to(e + p * B, (16,)).astype(
                        jnp.int32)
                    dval = plsc.load_gather(dv, [esplat])
                    for r in range(NREG):
                        sl = pl.ds(r * 16, 16)
                        sl2 = pl.ds(F + r * 16, 16)
                        out_v[e, sl] = (w12_v[e, sl] * ga_v[e, sl] * dval
                                        + w12_v[e, sl2] * gb_v[e, sl])

        def issue_scatter(s, par, p):
            out_v = data[s][3]
            for t in range(B // 16):
                idxs_v[s][pl.ds(t * 16, 16)] = \
                    idxi_o[par][pl.ds(p * B + t * 16, 16)]
            pltpu.async_copy(out_v, acc.at[idxs_v[s]], sem_scat[s], add=True)

        def wait_scatter(s):
            out_v = data[s][3]
            pltpu.make_async_copy(out_v, acc.at[idxs_v[s]],
                                  sem_scat[s]).wait()

        kend = chi_p * 2 * OCT

        issue_oct(0, kstart)
        wait_oct(0)
        issue_gathers(0, 0, 0, kstart)
        issue_gathers(1, 0, 1, kstart + 1)

        @pl.loop(clo_p, chi_p)
        def _pair(mm):
            for oo in (0, 1):
                par = oo
                k0 = (2 * mm + oo) * OCT
                for p in range(OCT):
                    k = k0 + p
                    s = p % 2

                    wait_loads(s, par, p)

                    @pl.when(k >= kstart + 2)
                    def _():
                        wait_scatter(s)

                    compute(s, par, p)
                    issue_scatter(s, par, p)

                    if p == 2:
                        @pl.when(k0 + OCT < kend)
                        def _():
                            issue_oct(1 - par, k0 + OCT)

                    if p == 6:
                        @pl.when(k0 + OCT < kend)
                        def _():
                            wait_oct(1 - par)

                    @pl.when(k + 2 < kend)
                    def _():
                        issue_gathers(s, par if p < 6 else 1 - par,
                                      (p + 2) % OCT, k + 2)

        wait_scatter(0)
        wait_scatter(1)
        init_writeout(out_hbm, outp_hbm, use_alt, rowoff, False)

    npair = nchunk // (2 * OCT)
    zero = jnp.zeros((), jnp.int32)
    false_ = zero > 0

    @pl.when(cid == 0)
    def _():
        run_pass(True, x0_hbm, x0_hbm, q2_hbm, outq_hbm, zero,
                 zero, npair, false_)

    dlo = jnp.where(cid == 0, 0, 1)
    dhi = jnp.where(cid == 0, 2, 3)

    @pl.loop(dlo, dhi)
    def _dmu(d):
        is_part = jnp.logical_and(cid == 0, d == 1)
        clo_p = jnp.where(jnp.logical_and(cid == 1, d == 1),
                          CSPLIT // (2 * OCT), 0)
        chi_p = jnp.where(is_part, CSPLIT // (2 * OCT), npair)
        run_pass(False, x1_hbm, pcat_hbm, mucat_hbm, outmu_hbm, d,
                 clo_p, chi_p, is_part)


_sc_edges = functools.partial(
    pl.kernel,
    out_type=[jax.ShapeDtypeStruct((N, F), jnp.float32),
              jax.ShapeDtypeStruct((3 * N, F), jnp.float32),
              jax.ShapeDtypeStruct((N, F), jnp.float32)],
    mesh=plsc.VectorSubcoreMesh(core_axis_name="c", subcore_axis_name="s"),
    compiler_params=pltpu.CompilerParams(needs_layout_passes=False),
    scratch_types=[pltpu.VMEM_SHARED((N, F), jnp.float32)]
    + [pltpu.VMEM((OCT * B,), jnp.int32)] * 2
    + [pltpu.VMEM((OCT * B,), jnp.int32)] * 2
    + [pltpu.VMEM((OCT * B,), jnp.int32)] * 2
    + [pltpu.VMEM((OCT * B,), jnp.float32)] * 2
    + [pltpu.VMEM((B,), jnp.int32)] * 2
    + [pltpu.VMEM((B, 2 * F), jnp.float32),
       pltpu.VMEM((B, F), jnp.float32),
       pltpu.VMEM((B, F), jnp.float32),
       pltpu.VMEM((B, F), jnp.float32)]
    + [pltpu.VMEM((B, 2 * F), jnp.float32),
       pltpu.VMEM((B, F), jnp.float32),
       pltpu.VMEM((B, F), jnp.float32),
       pltpu.VMEM((B, F), jnp.float32)]
    + [pltpu.SemaphoreType.DMA] * 2
    + [pltpu.SemaphoreType.DMA] * 4,
)(_sc_edge_kernel)


def _add_body(a_ref, b_ref, o_ref):
    o_ref[...] = a_ref[...] + b_ref[...]


def _padd(a, b):
    BN = 1000
    blk = pl.BlockSpec((BN, F), lambda i: (i, 0))
    return pl.pallas_call(
        _add_body,
        grid=(N // BN,),
        in_specs=[blk, blk],
        out_specs=blk,
        out_shape=jax.ShapeDtypeStruct((N, F), jnp.float32),
    )(a, b)


def kernel(q, mu, W_ij, dir_ij, pairlist, W1, b1, W2, b2):
    q2 = q[:, 0, :]
    mu0, mu1, mu2 = mu[:, 0, :], mu[:, 1, :], mu[:, 2, :]
    x0, x1, p0, p1, p2 = _mlp(q2, W1, b1, W2, b2, mu0, mu1, mu2)
    pcat = jnp.concatenate([p0, p1, p2], axis=0)
    mucat = jnp.concatenate([mu0, mu1, mu2], axis=0)
    dcat = jnp.concatenate([dir_ij[:, 0], dir_ij[:, 1], dir_ij[:, 2]], axis=0)
    zq = jnp.zeros((N, F), jnp.float32)
    idx_i, idx_j = pairlist[0], pairlist[1]
    outq, outmu, outp = _sc_edges(W_ij, x0, x1, pcat, dcat,
                                  q2, mucat, zq, idx_i, idx_j)
    m0 = outmu[:N]
    m1 = _padd(outmu[N:2 * N], outp)
    m2 = outmu[2 * N:]
    return outq[:, None, :], jnp.stack([m0, m1, m2], axis=1)
